# Initial kernel scaffold; baseline (speedup 1.0000x reference)
import jax
import jax.numpy as jnp
from jax.experimental import pallas as pl


def kernel(x, params):
    raise NotImplementedError("write your pallas kernel here")


# speedup vs baseline: 7.5060x; 7.5060x over previous
import functools

import jax
import jax.numpy as jnp
from jax.experimental import pallas as pl

_SA = [(1024, 0.1, 32, 12, [32, 32, 64]),
       (256, 0.2, 32, 67, [64, 64, 128]),
       (64, 0.4, 32, 131, [128, 128, 256]),
       (16, 0.8, 32, 259, [256, 256, 512])]
_FP = [(768, [256, 256]), (384, [256, 256]), (320, [256, 128]),
       (128, [128, 128, 128])]
_NS = 32
_F32 = jnp.float32


def _fold(layer):
    s = layer['g'] / jnp.sqrt(1.0 + 1e-05)
    return layer['W'] * s[None, :], (layer['b'] * s + layer['be'])[None, :]


def _lane_cumsum(x, n):
    sh = 1
    while sh < n:
        z = jnp.zeros(x.shape[:-1] + (sh,), x.dtype)
        x = x + jnp.concatenate([z, x[..., :-sh]], axis=-1)
        sh *= 2
    return x


def _fps_kernel(x_ref, y_ref, z_ref, out_ref, *, npoint, n):
    X = x_ref[...]
    Y = y_ref[...]
    Z = z_ref[...]
    bb = X.shape[0]
    niota = jax.lax.broadcasted_iota(jnp.int32, (bb, n), 1)
    piota = jax.lax.broadcasted_iota(jnp.int32, (bb, npoint), 1)

    def body(i, carry):
        dist_acc, far = carry
        out_ref[...] = jnp.where(piota == i, far + jnp.zeros_like(piota),
                                 out_ref[...])
        oh = (niota == far).astype(_F32)
        cx = jnp.sum(X * oh, axis=1, keepdims=True)
        cy = jnp.sum(Y * oh, axis=1, keepdims=True)
        cz = jnp.sum(Z * oh, axis=1, keepdims=True)
        dx = X - cx
        dy = Y - cy
        dz = Z - cz
        dist = (dx * dx + dy * dy) + dz * dz
        dist_acc = jnp.minimum(dist_acc, dist)
        m = jnp.max(dist_acc, axis=1, keepdims=True)
        far2 = jnp.min(jnp.where(dist_acc == m, niota, n), axis=1,
                       keepdims=True).astype(jnp.int32)
        return dist_acc, far2

    init = (jnp.full((bb, n), 1e10, _F32), jnp.zeros((bb, 1), jnp.int32))
    jax.lax.fori_loop(0, npoint, body, init)


def _fps(xyz, npoint):
    b, n, _ = xyz.shape
    fn = functools.partial(_fps_kernel, npoint=npoint, n=n)
    out = pl.pallas_call(
        fn,
        out_shape=jax.ShapeDtypeStruct((b, npoint), jnp.int32),
    )(xyz[:, :, 0], xyz[:, :, 1], xyz[:, :, 2])
    return out


def _sa_kernel(ptsf_ref, xt_ref, fidx_ref, *refs, chunk, n, cf, r2, nlayers):
    nx_ref, ft_ref = refs[-2:]
    wb = refs[:-2]
    pf = ptsf_ref[0]
    xt = xt_ref[0]
    fidx = fidx_ref[0]

    niota = jax.lax.broadcasted_iota(jnp.int32, (chunk, n), 1)
    oh = (niota == fidx).astype(_F32)
    nx = jnp.sum(oh * xt[0:1, :], axis=1, keepdims=True)
    ny = jnp.sum(oh * xt[1:2, :], axis=1, keepdims=True)
    nz = jnp.sum(oh * xt[2:3, :], axis=1, keepdims=True)
    new3 = jnp.concatenate([nx, ny, nz], axis=1)

    s2n = (nx * nx + ny * ny) + nz * nz
    s2d = (xt[0:1, :] * xt[0:1, :] + xt[1:2, :] * xt[1:2, :]) \
        + xt[2:3, :] * xt[2:3, :]
    d = -2.0 * jnp.dot(new3, xt, preferred_element_type=_F32)
    d = d + s2n
    d = d + s2d

    valid = jnp.logical_not(d > r2)
    c = _lane_cumsum(valid.astype(jnp.int32), n)
    sel = jnp.logical_and(valid, c <= _NS)
    cnt = jnp.sum(sel.astype(jnp.int32), axis=1, keepdims=True)

    gs = []
    for k in range(_NS):
        mk = jnp.logical_and(sel, c == k + 1).astype(_F32)
        gs.append(jnp.dot(mk, pf, preferred_element_type=_F32,
                          precision=jax.lax.Precision.HIGHEST)[None])
    G = jnp.concatenate(gs, axis=0)
    kio = jax.lax.broadcasted_iota(jnp.int32, (_NS, chunk, 1), 0)
    G = jnp.where(kio < cnt[None], G, G[0:1])
    nxpad = jnp.concatenate([new3, jnp.zeros((chunk, cf - 3), _F32)], axis=1)
    G = G - nxpad[None]

    h = G.reshape(_NS * chunk, cf)
    for li in range(nlayers):
        W = wb[2 * li][...]
        bias = wb[2 * li + 1][...]
        h = jnp.dot(h, W, preferred_element_type=_F32) + bias
        h = jnp.maximum(h, 0.0)
    cout = h.shape[-1]
    feat = jnp.max(h.reshape(_NS, chunk, cout), axis=0)

    nx_ref[0] = new3
    ft_ref[0] = feat


def _sa_call(ptsf, xyzT, fidx3, layers, npoint, radius):
    b, n, cf = ptsf.shape
    chunk = min(npoint, 128)
    nch = npoint // chunk
    cout = layers[-1][0].shape[1]
    nlayers = len(layers)
    wb = []
    for W, bias in layers:
        wb.append(W)
        wb.append(bias)
    full = lambda a: pl.BlockSpec(a.shape, lambda bi, si: (0,) * a.ndim)
    fn = functools.partial(_sa_kernel, chunk=chunk, n=n, cf=cf,
                           r2=radius * radius, nlayers=nlayers)
    new_xyz, feat = pl.pallas_call(
        fn,
        grid=(b, nch),
        in_specs=[
            pl.BlockSpec((1, n, cf), lambda bi, si: (bi, 0, 0)),
            pl.BlockSpec((1, 3, n), lambda bi, si: (bi, 0, 0)),
            pl.BlockSpec((1, chunk, 1), lambda bi, si: (bi, si, 0)),
        ] + [full(a) for a in wb],
        out_specs=[
            pl.BlockSpec((1, chunk, 3), lambda bi, si: (bi, si, 0)),
            pl.BlockSpec((1, chunk, cout), lambda bi, si: (bi, si, 0)),
        ],
        out_shape=[
            jax.ShapeDtypeStruct((b, npoint, 3), _F32),
            jax.ShapeDtypeStruct((b, npoint, cout), _F32),
        ],
    )(ptsf, xyzT, fidx3, *wb)
    return new_xyz, feat


def _fp_kernel(x1_ref, x2t_ref, p2_ref, *refs, chunk, m, c1, nlayers, head):
    out_ref = refs[-1]
    if c1:
        p1_ref = refs[0]
        wrefs = refs[1:-1]
    else:
        p1_ref = None
        wrefs = refs[:-1]

    x1c = x1_ref[0]
    x2t = x2t_ref[0]
    p2 = p2_ref[0]

    s2a = (x1c[:, 0:1] * x1c[:, 0:1] + x1c[:, 1:2] * x1c[:, 1:2]) \
        + x1c[:, 2:3] * x1c[:, 2:3]
    s2b = (x2t[0:1, :] * x2t[0:1, :] + x2t[1:2, :] * x2t[1:2, :]) \
        + x2t[2:3, :] * x2t[2:3, :]
    d = -2.0 * jnp.dot(x1c, x2t, preferred_element_type=_F32)
    d = d + s2a
    d = d + s2b

    miota = jax.lax.broadcasted_iota(jnp.int32, (chunk, m), 1)
    dd = d
    recs = []
    idxs = []
    for _ in range(3):
        mn = jnp.min(dd, axis=1, keepdims=True)
        ij = jnp.min(jnp.where(dd == mn, miota, m), axis=1, keepdims=True)
        idxs.append(ij)
        recs.append(1.0 / (mn + 1e-08))
        dd = jnp.where(miota == ij, 3e38, dd)
    norm = (recs[0] + recs[1]) + recs[2]
    acc = None
    for j in range(3):
        wj = recs[j] / norm
        ohw = jnp.where(miota == idxs[j], wj, 0.0)
        t = jnp.dot(ohw, p2, preferred_element_type=_F32,
                    precision=jax.lax.Precision.HIGHEST)
        acc = t if acc is None else acc + t

    if c1:
        p1c = p1_ref[0]
        h = (jnp.dot(p1c, wrefs[0][...], preferred_element_type=_F32)
             + jnp.dot(acc, wrefs[1][...], preferred_element_type=_F32)) \
            + wrefs[2][...]
        h = jnp.maximum(h, 0.0)
        base = 3
    else:
        h = jnp.dot(acc, wrefs[0][...], preferred_element_type=_F32) \
            + wrefs[1][...]
        h = jnp.maximum(h, 0.0)
        base = 2
    for li in range(1, nlayers):
        W = wrefs[base + 2 * (li - 1)][...]
        bias = wrefs[base + 2 * (li - 1) + 1][...]
        h = jnp.dot(h, W, preferred_element_type=_F32) + bias
        h = jnp.maximum(h, 0.0)
    if head:
        k = base + 2 * (nlayers - 1)
        h = jnp.dot(h, wrefs[k][...], preferred_element_type=_F32) \
            + wrefs[k + 1][...]
        h = jnp.maximum(h, 0.0)
        logits = jnp.dot(h, wrefs[k + 2][...], preferred_element_type=_F32) \
            + wrefs[k + 3][...]
        mx = jnp.max(logits, axis=-1, keepdims=True)
        s = logits - mx
        h = s - jnp.log(jnp.sum(jnp.exp(s), axis=-1, keepdims=True))
    out_ref[0] = h


def _fp_call(xyz1, xyz2T, pts1, pts2, layers, head_layers=None):
    b, n, _ = xyz1.shape
    m = xyz2T.shape[2]
    chunk = min(n, 128)
    nch = n // chunk
    c1 = pts1.shape[2] if pts1 is not None else 0
    nlayers = len(layers)
    wb = []
    if c1:
        w0, b0 = layers[0]
        wb += [w0[:c1], w0[c1:], b0]
    else:
        wb += [layers[0][0], layers[0][1]]
    for W, bias in layers[1:]:
        wb.append(W)
        wb.append(bias)
    if head_layers is not None:
        for W, bias in head_layers:
            wb.append(W)
            wb.append(bias)
        cout = head_layers[-1][0].shape[1]
    else:
        cout = layers[-1][0].shape[1]
    full = lambda a: pl.BlockSpec(a.shape, lambda bi, si: (0,) * a.ndim)
    ins = [xyz1, xyz2T, pts2]
    in_specs = [
        pl.BlockSpec((1, chunk, 3), lambda bi, si: (bi, si, 0)),
        pl.BlockSpec((1, 3, m), lambda bi, si: (bi, 0, 0)),
        pl.BlockSpec((1,) + pts2.shape[1:], lambda bi, si: (bi, 0, 0)),
    ]
    if c1:
        ins.append(pts1)
        in_specs.append(pl.BlockSpec((1, chunk, c1),
                                     lambda bi, si: (bi, si, 0)))
    ins += wb
    in_specs += [full(a) for a in wb]
    fn = functools.partial(_fp_kernel, chunk=chunk, m=m, c1=c1,
                           nlayers=nlayers, head=head_layers is not None)
    out = pl.pallas_call(
        fn,
        grid=(b, nch),
        in_specs=in_specs,
        out_specs=pl.BlockSpec((1, chunk, cout), lambda bi, si: (bi, si, 0)),
        out_shape=jax.ShapeDtypeStruct((b, n, cout), _F32),
    )(*ins)
    return out


def kernel(x, params):
    xyz0 = x[:, :, :3]
    level_xyz = [xyz0]
    level_pts = [x]
    xyz, pts = xyz0, x
    for name, (npoint, radius, nsample, cin, mlp) in zip(
            ['sa1', 'sa2', 'sa3', 'sa4'], _SA):
        layers = [_fold(l) for l in params[name]]
        ptsf = jnp.concatenate([xyz, pts], axis=-1)
        fidx = _fps(xyz, npoint)
        xyzT = jnp.swapaxes(xyz, 1, 2)
        new_xyz, feat = _sa_call(ptsf, xyzT, fidx[..., None], layers,
                                 npoint, radius)
        xyz, pts = new_xyz, feat
        level_xyz.append(xyz)
        level_pts.append(pts)

    for li, name in zip([3, 2, 1], ['fp4', 'fp3', 'fp2']):
        layers = [_fold(l) for l in params[name]]
        level_pts[li] = _fp_call(
            level_xyz[li], jnp.swapaxes(level_xyz[li + 1], 1, 2),
            level_pts[li], level_pts[li + 1], layers)

    fp1_layers = [_fold(l) for l in params['fp1']]
    head = [_fold(params['head1']),
            (params['head2']['W'], params['head2']['b'][None, :])]
    out = _fp_call(level_xyz[0], jnp.swapaxes(level_xyz[1], 1, 2),
                   None, level_pts[1], fp1_layers, head_layers=head)
    return out


# speedup vs baseline: 10.4405x; 1.3909x over previous
import functools

import jax
import jax.numpy as jnp
from jax.experimental import pallas as pl

_SA = [(1024, 0.1, 32, 12, [32, 32, 64]),
       (256, 0.2, 32, 67, [64, 64, 128]),
       (64, 0.4, 32, 131, [128, 128, 256]),
       (16, 0.8, 32, 259, [256, 256, 512])]
_FP = [(768, [256, 256]), (384, [256, 256]), (320, [256, 128]),
       (128, [128, 128, 128])]
_NS = 32
_F32 = jnp.float32


def _prep(layer):
    return (layer['W'], layer['b'][None, :], layer['g'][None, :],
            layer['be'][None, :])


def _bn_act(h, g, be):
    h = h * g / jnp.sqrt(1.0 + 1e-05) + be
    return jnp.maximum(h, 0.0)


def _split3(x):
    h1 = x.astype(jnp.bfloat16).astype(_F32)
    r = x - h1
    h2 = r.astype(jnp.bfloat16).astype(_F32)
    h3 = r - h2
    return h1, h2, h3


def _exact_gather(mk, parts):
    g = None
    for p in parts:
        t = jnp.dot(mk, p, preferred_element_type=_F32)
        g = t if g is None else g + t
    return g


def _lane_cumsum(x, n):
    sh = 1
    while sh < n:
        z = jnp.zeros(x.shape[:-1] + (sh,), x.dtype)
        x = x + jnp.concatenate([z, x[..., :-sh]], axis=-1)
        sh *= 2
    return x


def _fps_kernel(x_ref, y_ref, z_ref, out_ref, *, npoint, n):
    X = x_ref[...]
    Y = y_ref[...]
    Z = z_ref[...]
    bb = X.shape[0]
    niota = jax.lax.broadcasted_iota(jnp.int32, (bb, n), 1)
    piota = jax.lax.broadcasted_iota(jnp.int32, (bb, npoint), 1)

    def body(i, carry):
        dist_acc, far = carry
        out_ref[...] = jnp.where(piota == i, far + jnp.zeros_like(piota),
                                 out_ref[...])
        oh = (niota == far).astype(_F32)
        cx = jnp.sum(X * oh, axis=1, keepdims=True)
        cy = jnp.sum(Y * oh, axis=1, keepdims=True)
        cz = jnp.sum(Z * oh, axis=1, keepdims=True)
        dx = X - cx
        dy = Y - cy
        dz = Z - cz
        dist = (dx * dx + dy * dy) + dz * dz
        dist_acc = jnp.minimum(dist_acc, dist)
        m = jnp.max(dist_acc, axis=1, keepdims=True)
        far2 = jnp.min(jnp.where(dist_acc == m, niota, n), axis=1,
                       keepdims=True).astype(jnp.int32)
        return dist_acc, far2

    init = (jnp.full((bb, n), 1e10, _F32), jnp.zeros((bb, 1), jnp.int32))
    jax.lax.fori_loop(0, npoint, body, init)


def _fps(xyz, npoint):
    b, n, _ = xyz.shape
    fn = functools.partial(_fps_kernel, npoint=npoint, n=n)
    out = pl.pallas_call(
        fn,
        out_shape=jax.ShapeDtypeStruct((b, npoint), jnp.int32),
    )(xyz[:, :, 0], xyz[:, :, 1], xyz[:, :, 2])
    return out


def _sa_kernel(ptsf_ref, xt_ref, fidx_ref, *refs, chunk, n, cf, r2, nlayers):
    nx_ref, ft_ref = refs[-2:]
    wb = refs[:-2]
    pf = ptsf_ref[0]
    xt = xt_ref[0]
    fidx = fidx_ref[0]

    niota = jax.lax.broadcasted_iota(jnp.int32, (chunk, n), 1)
    oh = (niota == fidx).astype(_F32)
    nx = jnp.sum(oh * xt[0:1, :], axis=1, keepdims=True)
    ny = jnp.sum(oh * xt[1:2, :], axis=1, keepdims=True)
    nz = jnp.sum(oh * xt[2:3, :], axis=1, keepdims=True)
    new3 = jnp.concatenate([nx, ny, nz], axis=1)

    s2n = (nx * nx + ny * ny) + nz * nz
    s2d = (xt[0:1, :] * xt[0:1, :] + xt[1:2, :] * xt[1:2, :]) \
        + xt[2:3, :] * xt[2:3, :]
    d = -2.0 * jnp.dot(new3, xt, preferred_element_type=_F32)
    d = d + s2n
    d = d + s2d

    valid = jnp.logical_not(d > r2)
    c = _lane_cumsum(valid.astype(jnp.int32), n)
    sel = jnp.logical_and(valid, c <= _NS)
    cnt = jnp.sum(sel.astype(jnp.int32), axis=1, keepdims=True)

    pfp = _split3(pf)
    gs = []
    for k in range(_NS):
        mk = jnp.logical_and(sel, c == k + 1).astype(_F32)
        gs.append(_exact_gather(mk, pfp)[None])
    G = jnp.concatenate(gs, axis=0)
    kio = jax.lax.broadcasted_iota(jnp.int32, (_NS, chunk, 1), 0)
    glast = pf[n - 1:n, :][None] + jnp.zeros_like(G[0:1])
    pad = jnp.where((cnt == 0)[None], glast, G[0:1])
    G = jnp.where(kio < cnt[None], G, pad)
    nxpad = jnp.concatenate([new3, jnp.zeros((chunk, cf - 3), _F32)], axis=1)
    G = G - nxpad[None]

    h = G.reshape(_NS * chunk, cf)
    for li in range(nlayers):
        W, bias, g, be = (wb[4 * li][...], wb[4 * li + 1][...],
                          wb[4 * li + 2][...], wb[4 * li + 3][...])
        h = jnp.dot(h, W, preferred_element_type=_F32) + bias
        h = _bn_act(h, g, be)
    cout = h.shape[-1]
    feat = jnp.max(h.reshape(_NS, chunk, cout), axis=0)

    nx_ref[0] = new3
    ft_ref[0] = feat


def _sa_call(ptsf, xyzT, fidx3, layers, npoint, radius):
    b, n, cf = ptsf.shape
    chunk = min(npoint, 128)
    nch = npoint // chunk
    cout = layers[-1][0].shape[1]
    nlayers = len(layers)
    wb = [a for lay in layers for a in lay]
    full = lambda a: pl.BlockSpec(a.shape, lambda bi, si: (0,) * a.ndim)
    fn = functools.partial(_sa_kernel, chunk=chunk, n=n, cf=cf,
                           r2=radius * radius, nlayers=nlayers)
    new_xyz, feat = pl.pallas_call(
        fn,
        grid=(b, nch),
        in_specs=[
            pl.BlockSpec((1, n, cf), lambda bi, si: (bi, 0, 0)),
            pl.BlockSpec((1, 3, n), lambda bi, si: (bi, 0, 0)),
            pl.BlockSpec((1, chunk, 1), lambda bi, si: (bi, si, 0)),
        ] + [full(a) for a in wb],
        out_specs=[
            pl.BlockSpec((1, chunk, 3), lambda bi, si: (bi, si, 0)),
            pl.BlockSpec((1, chunk, cout), lambda bi, si: (bi, si, 0)),
        ],
        out_shape=[
            jax.ShapeDtypeStruct((b, npoint, 3), _F32),
            jax.ShapeDtypeStruct((b, npoint, cout), _F32),
        ],
    )(ptsf, xyzT, fidx3, *wb)
    return new_xyz, feat


def _fp_kernel(x1_ref, x2t_ref, p2_ref, *refs, chunk, m, c1, nlayers, head):
    out_ref = refs[-1]
    if c1:
        p1_ref = refs[0]
        wrefs = refs[1:-1]
    else:
        p1_ref = None
        wrefs = refs[:-1]

    x1c = x1_ref[0]
    x2t = x2t_ref[0]
    p2 = p2_ref[0]

    s2a = (x1c[:, 0:1] * x1c[:, 0:1] + x1c[:, 1:2] * x1c[:, 1:2]) \
        + x1c[:, 2:3] * x1c[:, 2:3]
    s2b = (x2t[0:1, :] * x2t[0:1, :] + x2t[1:2, :] * x2t[1:2, :]) \
        + x2t[2:3, :] * x2t[2:3, :]
    d = -2.0 * jnp.dot(x1c, x2t, preferred_element_type=_F32)
    d = d + s2a
    d = d + s2b

    miota = jax.lax.broadcasted_iota(jnp.int32, (chunk, m), 1)
    dd = d
    recs = []
    idxs = []
    for _ in range(3):
        mn = jnp.min(dd, axis=1, keepdims=True)
        ij = jnp.min(jnp.where(dd == mn, miota, m), axis=1, keepdims=True)
        idxs.append(ij)
        recs.append(1.0 / (mn + 1e-08))
        dd = jnp.where(miota == ij, 3e38, dd)
    norm = (recs[0] + recs[1]) + recs[2]
    p2p = _split3(p2)
    acc = None
    for j in range(3):
        wj = recs[j] / norm
        ohj = (miota == idxs[j]).astype(_F32)
        t = _exact_gather(ohj, p2p) * wj
        acc = t if acc is None else acc + t

    if c1:
        p1c = p1_ref[0]
        h = (jnp.dot(p1c, wrefs[0][...], preferred_element_type=_F32)
             + jnp.dot(acc, wrefs[1][...], preferred_element_type=_F32)) \
            + wrefs[2][...]
        h = _bn_act(h, wrefs[3][...], wrefs[4][...])
        base = 5
    else:
        h = jnp.dot(acc, wrefs[0][...], preferred_element_type=_F32) \
            + wrefs[1][...]
        h = _bn_act(h, wrefs[2][...], wrefs[3][...])
        base = 4
    for li in range(1, nlayers):
        W, bias, g, be = (wrefs[base + 4 * (li - 1)][...],
                          wrefs[base + 4 * (li - 1) + 1][...],
                          wrefs[base + 4 * (li - 1) + 2][...],
                          wrefs[base + 4 * (li - 1) + 3][...])
        h = jnp.dot(h, W, preferred_element_type=_F32) + bias
        h = _bn_act(h, g, be)
    if head:
        k = base + 4 * (nlayers - 1)
        h = jnp.dot(h, wrefs[k][...], preferred_element_type=_F32) \
            + wrefs[k + 1][...]
        h = _bn_act(h, wrefs[k + 2][...], wrefs[k + 3][...])
        logits = jnp.dot(h, wrefs[k + 4][...], preferred_element_type=_F32) \
            + wrefs[k + 5][...]
        mx = jnp.max(logits, axis=-1, keepdims=True)
        s = logits - mx
        h = s - jnp.log(jnp.sum(jnp.exp(s), axis=-1, keepdims=True))
    out_ref[0] = h


def _fp_call(xyz1, xyz2T, pts1, pts2, layers, head_layers=None):
    b, n, _ = xyz1.shape
    m = xyz2T.shape[2]
    chunk = min(n, 128)
    nch = n // chunk
    c1 = pts1.shape[2] if pts1 is not None else 0
    nlayers = len(layers)
    w0, b0, g0, be0 = layers[0]
    if c1:
        wb = [w0[:c1], w0[c1:], b0, g0, be0]
    else:
        wb = [w0, b0, g0, be0]
    for lay in layers[1:]:
        wb += list(lay)
    if head_layers is not None:
        wb += list(head_layers[0])
        wb += list(head_layers[1])
        cout = head_layers[1][0].shape[1]
    else:
        cout = layers[-1][0].shape[1]
    full = lambda a: pl.BlockSpec(a.shape, lambda bi, si: (0,) * a.ndim)
    ins = [xyz1, xyz2T, pts2]
    in_specs = [
        pl.BlockSpec((1, chunk, 3), lambda bi, si: (bi, si, 0)),
        pl.BlockSpec((1, 3, m), lambda bi, si: (bi, 0, 0)),
        pl.BlockSpec((1,) + pts2.shape[1:], lambda bi, si: (bi, 0, 0)),
    ]
    if c1:
        ins.append(pts1)
        in_specs.append(pl.BlockSpec((1, chunk, c1),
                                     lambda bi, si: (bi, si, 0)))
    ins += wb
    in_specs += [full(a) for a in wb]
    fn = functools.partial(_fp_kernel, chunk=chunk, m=m, c1=c1,
                           nlayers=nlayers, head=head_layers is not None)
    out = pl.pallas_call(
        fn,
        grid=(b, nch),
        in_specs=in_specs,
        out_specs=pl.BlockSpec((1, chunk, cout), lambda bi, si: (bi, si, 0)),
        out_shape=jax.ShapeDtypeStruct((b, n, cout), _F32),
    )(*ins)
    return out


def kernel(x, params):
    xyz0 = x[:, :, :3]
    level_xyz = [xyz0]
    level_pts = [x]
    xyz, pts = xyz0, x
    for name, (npoint, radius, nsample, cin, mlp) in zip(
            ['sa1', 'sa2', 'sa3', 'sa4'], _SA):
        layers = [_prep(l) for l in params[name]]
        ptsf = jnp.concatenate([xyz, pts], axis=-1)
        fidx = _fps(xyz, npoint)
        xyzT = jnp.swapaxes(xyz, 1, 2)
        new_xyz, feat = _sa_call(ptsf, xyzT, fidx[..., None], layers,
                                 npoint, radius)
        xyz, pts = new_xyz, feat
        level_xyz.append(xyz)
        level_pts.append(pts)

    for li, name in zip([3, 2, 1], ['fp4', 'fp3', 'fp2']):
        layers = [_prep(l) for l in params[name]]
        level_pts[li] = _fp_call(
            level_xyz[li], jnp.swapaxes(level_xyz[li + 1], 1, 2),
            level_pts[li], level_pts[li + 1], layers)

    fp1_layers = [_prep(l) for l in params['fp1']]
    head = [_prep(params['head1']),
            (params['head2']['W'], params['head2']['b'][None, :])]
    out = _fp_call(level_xyz[0], jnp.swapaxes(level_xyz[1], 1, 2),
                   None, level_pts[1], fp1_layers, head_layers=head)
    return out


# speedup vs baseline: 12.3377x; 1.1817x over previous
import functools

import jax
import jax.numpy as jnp
from jax.experimental import pallas as pl

_SA = [(1024, 0.1, 32, 12, [32, 32, 64]),
       (256, 0.2, 32, 67, [64, 64, 128]),
       (64, 0.4, 32, 131, [128, 128, 256]),
       (16, 0.8, 32, 259, [256, 256, 512])]
_FP = [(768, [256, 256]), (384, [256, 256]), (320, [256, 128]),
       (128, [128, 128, 128])]
_NS = 32
_F32 = jnp.float32


def _prep(layer):
    return (layer['W'], layer['b'][None, :], layer['g'][None, :],
            layer['be'][None, :])


def _bn_act(h, g, be):
    h = h * g / jnp.sqrt(1.0 + 1e-05) + be
    return jnp.maximum(h, 0.0)


def _split3(x):
    h1 = x.astype(jnp.bfloat16).astype(_F32)
    r = x - h1
    h2 = r.astype(jnp.bfloat16).astype(_F32)
    h3 = r - h2
    return h1, h2, h3


def _exact_gather(mk, parts):
    g = None
    for p in parts:
        t = jnp.dot(mk, p, preferred_element_type=_F32)
        g = t if g is None else g + t
    return g


def _lane_cumsum(x, n):
    sh = 1
    while sh < n:
        z = jnp.zeros(x.shape[:-1] + (sh,), x.dtype)
        x = x + jnp.concatenate([z, x[..., :-sh]], axis=-1)
        sh *= 2
    return x


def _fps_kernel(x_ref, y_ref, z_ref, out_ref, *, npoint, n):
    X = x_ref[...]
    Y = y_ref[...]
    Z = z_ref[...]
    bb = X.shape[0]
    niota = jax.lax.broadcasted_iota(jnp.int32, (bb, n), 1)
    piota = jax.lax.broadcasted_iota(jnp.int32, (bb, npoint), 1)

    def body(i, carry):
        dist_acc, far = carry
        out_ref[...] = jnp.where(piota == i, far + jnp.zeros_like(piota),
                                 out_ref[...])
        oh = (niota == far).astype(_F32)
        cx = jnp.sum(X * oh, axis=1, keepdims=True)
        cy = jnp.sum(Y * oh, axis=1, keepdims=True)
        cz = jnp.sum(Z * oh, axis=1, keepdims=True)
        dx = X - cx
        dy = Y - cy
        dz = Z - cz
        dist = (dx * dx + dy * dy) + dz * dz
        dist_acc = jnp.minimum(dist_acc, dist)
        m = jnp.max(dist_acc, axis=1, keepdims=True)
        far2 = jnp.min(jnp.where(dist_acc == m, niota, n), axis=1,
                       keepdims=True).astype(jnp.int32)
        return dist_acc, far2

    init = (jnp.full((bb, n), 1e10, _F32), jnp.zeros((bb, 1), jnp.int32))
    jax.lax.fori_loop(0, npoint, body, init)


def _fps(xyz, npoint):
    b, n, _ = xyz.shape
    fn = functools.partial(_fps_kernel, npoint=npoint, n=n)
    out = pl.pallas_call(
        fn,
        out_shape=jax.ShapeDtypeStruct((b, npoint), jnp.int32),
    )(xyz[:, :, 0], xyz[:, :, 1], xyz[:, :, 2])
    return out


def _sa_kernel(pf1_ref, pf2_ref, pf3_ref, xt_ref, fidx_ref, *refs,
               chunk, n, cf, r2, nlayers):
    nx_ref, ft_ref = refs[-2:]
    wb = refs[:-2]
    pfp = (pf1_ref[0], pf2_ref[0], pf3_ref[0])
    xt = xt_ref[0]
    fidx = fidx_ref[0]

    niota = jax.lax.broadcasted_iota(jnp.int32, (chunk, n), 1)
    oh = (niota == fidx).astype(_F32)
    nx = jnp.sum(oh * xt[0:1, :], axis=1, keepdims=True)
    ny = jnp.sum(oh * xt[1:2, :], axis=1, keepdims=True)
    nz = jnp.sum(oh * xt[2:3, :], axis=1, keepdims=True)
    new3 = jnp.concatenate([nx, ny, nz], axis=1)

    s2n = (nx * nx + ny * ny) + nz * nz
    s2d = (xt[0:1, :] * xt[0:1, :] + xt[1:2, :] * xt[1:2, :]) \
        + xt[2:3, :] * xt[2:3, :]
    d = -2.0 * jnp.dot(new3, xt, preferred_element_type=_F32)
    d = d + s2n
    d = d + s2d

    valid = jnp.logical_not(d > r2)
    c = _lane_cumsum(valid.astype(jnp.int32), n)
    sel = jnp.logical_and(valid, c <= _NS)
    cnt = jnp.sum(sel.astype(jnp.int32), axis=1, keepdims=True)
    csel = jnp.where(sel, c, 0)

    gs = []
    for k in range(_NS):
        mk = (csel == k + 1).astype(_F32)
        gs.append(_exact_gather(mk, pfp)[None])
    G = jnp.concatenate(gs, axis=0)
    kio = jax.lax.broadcasted_iota(jnp.int32, (_NS, chunk, 1), 0)
    plast = (pfp[0][n - 1:n, :] + pfp[1][n - 1:n, :]) + pfp[2][n - 1:n, :]
    glast = plast[None] + jnp.zeros_like(G[0:1])
    pad = jnp.where((cnt == 0)[None], glast, G[0:1])
    G = jnp.where(kio < cnt[None], G, pad)
    nxpad = jnp.concatenate([new3, jnp.zeros((chunk, cf - 3), _F32)], axis=1)
    G = G - nxpad[None]

    h = G.reshape(_NS * chunk, cf)
    for li in range(nlayers):
        W, bias, g, be = (wb[4 * li][...], wb[4 * li + 1][...],
                          wb[4 * li + 2][...], wb[4 * li + 3][...])
        h = jnp.dot(h, W, preferred_element_type=_F32) + bias
        h = _bn_act(h, g, be)
    cout = h.shape[-1]
    feat = jnp.max(h.reshape(_NS, chunk, cout), axis=0)

    nx_ref[0] = new3
    ft_ref[0] = feat


def _sa_call(ptsf, xyzT, fidx3, layers, npoint, radius):
    b, n, cf = ptsf.shape
    chunk = min(npoint, 128)
    nch = npoint // chunk
    cout = layers[-1][0].shape[1]
    nlayers = len(layers)
    wb = [a for lay in layers for a in lay]
    full = lambda a: pl.BlockSpec(a.shape, lambda bi, si: (0,) * a.ndim)
    fn = functools.partial(_sa_kernel, chunk=chunk, n=n, cf=cf,
                           r2=radius * radius, nlayers=nlayers)
    pf1, pf2, pf3 = _split3(ptsf)
    new_xyz, feat = pl.pallas_call(
        fn,
        grid=(b, nch),
        in_specs=[
            pl.BlockSpec((1, n, cf), lambda bi, si: (bi, 0, 0)),
            pl.BlockSpec((1, n, cf), lambda bi, si: (bi, 0, 0)),
            pl.BlockSpec((1, n, cf), lambda bi, si: (bi, 0, 0)),
            pl.BlockSpec((1, 3, n), lambda bi, si: (bi, 0, 0)),
            pl.BlockSpec((1, chunk, 1), lambda bi, si: (bi, si, 0)),
        ] + [full(a) for a in wb],
        out_specs=[
            pl.BlockSpec((1, chunk, 3), lambda bi, si: (bi, si, 0)),
            pl.BlockSpec((1, chunk, cout), lambda bi, si: (bi, si, 0)),
        ],
        out_shape=[
            jax.ShapeDtypeStruct((b, npoint, 3), _F32),
            jax.ShapeDtypeStruct((b, npoint, cout), _F32),
        ],
    )(pf1, pf2, pf3, xyzT, fidx3, *wb)
    return new_xyz, feat


def _fp_kernel(x1_ref, x2t_ref, p2_ref, *refs, chunk, m, c1, nlayers, head):
    out_ref = refs[-1]
    if c1:
        p1_ref = refs[0]
        wrefs = refs[1:-1]
    else:
        p1_ref = None
        wrefs = refs[:-1]

    x1c = x1_ref[0]
    x2t = x2t_ref[0]
    p2 = p2_ref[0]

    s2a = (x1c[:, 0:1] * x1c[:, 0:1] + x1c[:, 1:2] * x1c[:, 1:2]) \
        + x1c[:, 2:3] * x1c[:, 2:3]
    s2b = (x2t[0:1, :] * x2t[0:1, :] + x2t[1:2, :] * x2t[1:2, :]) \
        + x2t[2:3, :] * x2t[2:3, :]
    d = -2.0 * jnp.dot(x1c, x2t, preferred_element_type=_F32)
    d = d + s2a
    d = d + s2b

    miota = jax.lax.broadcasted_iota(jnp.int32, (chunk, m), 1)
    dd = d
    recs = []
    idxs = []
    for _ in range(3):
        mn = jnp.min(dd, axis=1, keepdims=True)
        ij = jnp.min(jnp.where(dd == mn, miota, m), axis=1, keepdims=True)
        idxs.append(ij)
        recs.append(1.0 / (mn + 1e-08))
        dd = jnp.where(miota == ij, 3e38, dd)
    norm = (recs[0] + recs[1]) + recs[2]
    p2p = _split3(p2)
    acc = None
    for j in range(3):
        wj = recs[j] / norm
        ohj = (miota == idxs[j]).astype(_F32)
        t = _exact_gather(ohj, p2p) * wj
        acc = t if acc is None else acc + t

    if c1:
        p1c = p1_ref[0]
        h = (jnp.dot(p1c, wrefs[0][...], preferred_element_type=_F32)
             + jnp.dot(acc, wrefs[1][...], preferred_element_type=_F32)) \
            + wrefs[2][...]
        h = _bn_act(h, wrefs[3][...], wrefs[4][...])
        base = 5
    else:
        h = jnp.dot(acc, wrefs[0][...], preferred_element_type=_F32) \
            + wrefs[1][...]
        h = _bn_act(h, wrefs[2][...], wrefs[3][...])
        base = 4
    for li in range(1, nlayers):
        W, bias, g, be = (wrefs[base + 4 * (li - 1)][...],
                          wrefs[base + 4 * (li - 1) + 1][...],
                          wrefs[base + 4 * (li - 1) + 2][...],
                          wrefs[base + 4 * (li - 1) + 3][...])
        h = jnp.dot(h, W, preferred_element_type=_F32) + bias
        h = _bn_act(h, g, be)
    if head:
        k = base + 4 * (nlayers - 1)
        h = jnp.dot(h, wrefs[k][...], preferred_element_type=_F32) \
            + wrefs[k + 1][...]
        h = _bn_act(h, wrefs[k + 2][...], wrefs[k + 3][...])
        logits = jnp.dot(h, wrefs[k + 4][...], preferred_element_type=_F32) \
            + wrefs[k + 5][...]
        mx = jnp.max(logits, axis=-1, keepdims=True)
        s = logits - mx
        h = s - jnp.log(jnp.sum(jnp.exp(s), axis=-1, keepdims=True))
    out_ref[0] = h


def _fp_call(xyz1, xyz2T, pts1, pts2, layers, head_layers=None):
    b, n, _ = xyz1.shape
    m = xyz2T.shape[2]
    chunk = min(n, 128)
    nch = n // chunk
    c1 = pts1.shape[2] if pts1 is not None else 0
    nlayers = len(layers)
    w0, b0, g0, be0 = layers[0]
    if c1:
        wb = [w0[:c1], w0[c1:], b0, g0, be0]
    else:
        wb = [w0, b0, g0, be0]
    for lay in layers[1:]:
        wb += list(lay)
    if head_layers is not None:
        wb += list(head_layers[0])
        wb += list(head_layers[1])
        cout = head_layers[1][0].shape[1]
    else:
        cout = layers[-1][0].shape[1]
    full = lambda a: pl.BlockSpec(a.shape, lambda bi, si: (0,) * a.ndim)
    ins = [xyz1, xyz2T, pts2]
    in_specs = [
        pl.BlockSpec((1, chunk, 3), lambda bi, si: (bi, si, 0)),
        pl.BlockSpec((1, 3, m), lambda bi, si: (bi, 0, 0)),
        pl.BlockSpec((1,) + pts2.shape[1:], lambda bi, si: (bi, 0, 0)),
    ]
    if c1:
        ins.append(pts1)
        in_specs.append(pl.BlockSpec((1, chunk, c1),
                                     lambda bi, si: (bi, si, 0)))
    ins += wb
    in_specs += [full(a) for a in wb]
    fn = functools.partial(_fp_kernel, chunk=chunk, m=m, c1=c1,
                           nlayers=nlayers, head=head_layers is not None)
    out = pl.pallas_call(
        fn,
        grid=(b, nch),
        in_specs=in_specs,
        out_specs=pl.BlockSpec((1, chunk, cout), lambda bi, si: (bi, si, 0)),
        out_shape=jax.ShapeDtypeStruct((b, n, cout), _F32),
    )(*ins)
    return out


def kernel(x, params):
    xyz0 = x[:, :, :3]
    level_xyz = [xyz0]
    level_pts = [x]
    xyz, pts = xyz0, x
    for name, (npoint, radius, nsample, cin, mlp) in zip(
            ['sa1', 'sa2', 'sa3', 'sa4'], _SA):
        layers = [_prep(l) for l in params[name]]
        ptsf = jnp.concatenate([xyz, pts], axis=-1)
        fidx = _fps(xyz, npoint)
        xyzT = jnp.swapaxes(xyz, 1, 2)
        new_xyz, feat = _sa_call(ptsf, xyzT, fidx[..., None], layers,
                                 npoint, radius)
        xyz, pts = new_xyz, feat
        level_xyz.append(xyz)
        level_pts.append(pts)

    for li, name in zip([3, 2, 1], ['fp4', 'fp3', 'fp2']):
        layers = [_prep(l) for l in params[name]]
        level_pts[li] = _fp_call(
            level_xyz[li], jnp.swapaxes(level_xyz[li + 1], 1, 2),
            level_pts[li], level_pts[li + 1], layers)

    fp1_layers = [_prep(l) for l in params['fp1']]
    head = [_prep(params['head1']),
            (params['head2']['W'], params['head2']['b'][None, :])]
    out = _fp_call(level_xyz[0], jnp.swapaxes(level_xyz[1], 1, 2),
                   None, level_pts[1], fp1_layers, head_layers=head)
    return out
